# baseline (speedup 1.0000x reference)
import functools

import jax
import jax.numpy as jnp
from jax import lax
from jax.experimental import pallas as pl
from jax.experimental.pallas import tpu as pltpu
from jax.experimental.pallas import tpu_sc as plsc

_B = 16384
_D = 32
_NC = 2
_NS = 16
_NW = _NC * _NS
_BPW = _B // _NW


def _build():
    mesh = plsc.VectorSubcoreMesh(core_axis_name="c", subcore_axis_name="s")

    @functools.partial(
        pl.kernel,
        mesh=mesh,
        out_type=jax.ShapeDtypeStruct((_B, _D), jnp.float32),
        scratch_types=[
            pltpu.VMEM((_BPW,), jnp.int32),
            pltpu.SemaphoreType.DMA,
        ],
    )
    def gather_kernel(idx_hbm, table_hbm, out_hbm, idx_v, sem):
        wid = lax.axis_index("s") * _NC + lax.axis_index("c")
        base = wid * _BPW
        pltpu.sync_copy(idx_hbm.at[pl.ds(base, _BPW)], idx_v)

        def body(g, _):
            v = idx_v[pl.ds(g * 16, 16)]
            for l in range(16):
                pltpu.async_copy(
                    table_hbm.at[pl.ds(v[l], 1)],
                    out_hbm.at[pl.ds(base + g * 16 + l, 1)],
                    sem,
                )
            return 0

        lax.fori_loop(0, _BPW // 16, body, 0)
        pltpu.make_async_copy(
            table_hbm.at[pl.ds(0, _BPW)],
            out_hbm.at[pl.ds(base, _BPW)],
            sem,
        ).wait()

    return gather_kernel


def kernel(i, data):
    return _build()(i.astype(jnp.int32), data)


# speedup vs baseline: 1.7892x; 1.7892x over previous
import functools

import jax
import jax.numpy as jnp
from jax import lax
from jax.experimental import pallas as pl
from jax.experimental.pallas import tpu as pltpu
from jax.experimental.pallas import tpu_sc as plsc

_B = 16384
_D = 32
_NC = 2
_NS = 16
_NW = _NC * _NS
_BPW = _B // _NW
_L = 16
_NSEM = 8


def _build():
    mesh = plsc.VectorSubcoreMesh(core_axis_name="c", subcore_axis_name="s")

    @functools.partial(
        pl.kernel,
        mesh=mesh,
        out_type=jax.ShapeDtypeStruct((_B, _D), jnp.float32),
        scratch_types=[
            pltpu.VMEM((_BPW,), jnp.int32),
            pltpu.VMEM((_BPW, _D), jnp.float32),
        ]
        + [pltpu.SemaphoreType.DMA] * _NSEM,
    )
    def gather_kernel(idx_hbm, table_hbm, out_hbm, idx_v, rows_v, *sems):
        wid = lax.axis_index("s") * _NC + lax.axis_index("c")
        base = wid * _BPW
        pltpu.sync_copy(idx_hbm.at[pl.ds(base, _BPW)], idx_v)

        def body(g, _):
            v = idx_v[pl.ds(g * _L, _L)]
            for l in range(_L):
                pltpu.async_copy(
                    table_hbm.at[pl.ds(v[l], 1)],
                    rows_v.at[pl.ds(g * _L + l, 1)],
                    sems[l % _NSEM],
                )
            return 0

        lax.fori_loop(0, _BPW // _L, body, 0)
        for s in range(_NSEM):
            pltpu.make_async_copy(
                table_hbm.at[pl.ds(0, _BPW // _NSEM)],
                rows_v.at[pl.ds(s * (_BPW // _NSEM), _BPW // _NSEM)],
                sems[s],
            ).wait()
        pltpu.sync_copy(rows_v, out_hbm.at[pl.ds(base, _BPW)])

    return gather_kernel


def kernel(i, data):
    return _build()(i.astype(jnp.int32), data)
